# baseline (speedup 1.0000x reference)
import functools

import jax
import jax.numpy as jnp
from jax import lax
from jax.experimental import pallas as pl
from jax.experimental.pallas import tpu as pltpu
from jax.experimental.pallas import tpu_sc as plsc

N_JOBS = 6000
N_SKILLS = 3000
N_COMP = 1000
N = N_JOBS + N_SKILLS + N_COMP
D = 128
DH = 64
NL = 3
E = 320000
B = 4096

NC = 2
NS = 16
LANES = 16

NPAD = 10240
EPT = E // NS
EK = 160
NECH = EPT // EK
NPT = NPAD // NS
NSUB = 160
NSC = NPT // NSUB
BPT = B // NS
BH = 128
DEGW = 16

F32 = jnp.float32
I32 = jnp.int32


def _zero_rows(buf, nrows, ncol16):
    z = jnp.zeros((LANES,), F32)

    def body(i, _):
        for k in range(ncol16):
            buf[i, pl.ds(k * LANES, LANES)] = z
        return 0

    lax.fori_loop(0, nrows, body, 0)


def _add_offset(idx_ref, n, off):
    def body(j, _):
        base = j * LANES
        idx_ref[pl.ds(base, LANES)] = idx_ref[pl.ds(base, LANES)] + off
        return 0

    lax.fori_loop(0, n // LANES, body, 0)


def _sc_body(x0s, row_h, col_h, jobs_h, pos_h, neg_h,
             fin_h, jg_h, pg_h, ng_h,
             S, DEG, XP,
             degb, sbuf, fbuf, ridx, cidx, gidx,
             sem):
    c = lax.axis_index("c")
    s = lax.axis_index("s")
    nb = s * NPT
    xoff = c * NPAD

    _zero_rows(sbuf, NSUB, DH // LANES)
    _zero_rows(degb, NPT, DEGW // LANES)
    pltpu.sync_copy(degb, DEG.at[pl.ds(nb, NPT)])
    for sub in range(NSC):
        pltpu.sync_copy(sbuf, S.at[pl.ds(nb + sub * NSUB, NSUB)])
    one16 = jnp.ones((LANES,), F32)

    def fill_ones(i, _):
        degb[i, pl.ds(0, LANES)] = one16
        return 0

    lax.fori_loop(0, NPT, fill_ones, 0)
    plsc.subcore_barrier()

    def deg_chunk(ch, _):
        base = s * EPT + ch * EK
        pltpu.sync_copy(col_h.at[pl.ds(base, EK)], cidx)
        pltpu.sync_copy(degb.at[pl.ds(0, EK)], DEG.at[cidx], add=True)
        return 0

    lax.fori_loop(0, NECH, deg_chunk, 0)
    plsc.subcore_barrier()

    pltpu.sync_copy(DEG.at[pl.ds(nb, NPT)], degb)

    def dinv_row(n, _):
        d = degb[n, pl.ds(0, LANES)]
        ib = plsc.bitcast(d, I32)
        ib = jnp.int32(0x5F3759DF) - (ib >> 1)
        y = plsc.bitcast(ib, F32)
        xh = 0.5 * d
        y = y * (1.5 - xh * y * y)
        y = y * (1.5 - xh * y * y)
        y = y * (1.5 - xh * y * y)
        y = jnp.where(d > 0.5, y, 0.0)
        degb[n, pl.ds(0, LANES)] = y
        return 0

    lax.fori_loop(0, NPT, dinv_row, 0)

    for sub in range(NSC):
        base = nb + sub * NSUB
        pltpu.sync_copy(x0s.at[pl.ds(xoff + base, NSUB)], sbuf)
        pltpu.sync_copy(sbuf, fin_h.at[pl.ds(xoff + base, NSUB)])

        def scale_row(n, _):
            dv = degb[sub * NSUB + n, pl.ds(0, LANES)]
            for k in range(DH // LANES):
                ds = pl.ds(k * LANES, LANES)
                sbuf[n, ds] = dv * sbuf[n, ds]
            return 0

        lax.fori_loop(0, NSUB, scale_row, 0)
        pltpu.sync_copy(sbuf, XP.at[pl.ds(base, NSUB)])
    plsc.subcore_barrier()

    for layer in range(NL):

        def edge_chunk(ch, _):
            base = s * EPT + ch * EK
            pltpu.sync_copy(row_h.at[pl.ds(base, EK)], ridx)
            pltpu.sync_copy(col_h.at[pl.ds(base, EK)], cidx)
            pltpu.async_copy(XP.at[ridx], fbuf, sem).wait()
            pltpu.sync_copy(fbuf, S.at[cidx], add=True)
            return 0

        lax.fori_loop(0, NECH, edge_chunk, 0)
        plsc.subcore_barrier()

        for sub in range(NSC):
            base = nb + sub * NSUB
            pltpu.sync_copy(S.at[pl.ds(base, NSUB)], sbuf)
            pltpu.sync_copy(fin_h.at[pl.ds(xoff + base, NSUB)], fbuf)

            def node_row(n, _):
                dv = degb[sub * NSUB + n, pl.ds(0, LANES)]
                dv2 = dv * dv
                for k in range(DH // LANES):
                    ds = pl.ds(k * LANES, LANES)
                    sl = sbuf[n, ds]
                    fbuf[n, ds] = fbuf[n, ds] + dv * sl
                    sbuf[n, ds] = dv2 * sl
                return 0

            lax.fori_loop(0, NSUB, node_row, 0)
            pltpu.sync_copy(fbuf, fin_h.at[pl.ds(xoff + base, NSUB)])
            if layer < NL - 1:
                pltpu.sync_copy(sbuf, XP.at[pl.ds(base, NSUB)])
                _zero_rows(sbuf, NSUB, DH // LANES)
                pltpu.sync_copy(sbuf, S.at[pl.ds(base, NSUB)])
        plsc.subcore_barrier()

    for idx_h, out_h in ((jobs_h, jg_h), (pos_h, pg_h), (neg_h, ng_h)):
        bb = s * BPT
        pltpu.sync_copy(idx_h.at[pl.ds(bb, BPT)], gidx)
        _add_offset(gidx, BPT, xoff)
        for h in range(BPT // BH):
            pltpu.async_copy(fin_h.at[gidx.at[pl.ds(h * BH, BH)]],
                             fbuf.at[pl.ds(0, BH)], sem).wait()
            pltpu.sync_copy(fbuf.at[pl.ds(0, BH)],
                            out_h.at[pl.ds(c * B + bb + h * BH, BH)])


_sc_kernel = functools.partial(
    pl.kernel,
    out_type=(
        jax.ShapeDtypeStruct((2 * NPAD, DH), F32),
        jax.ShapeDtypeStruct((2 * B, DH), F32),
        jax.ShapeDtypeStruct((2 * B, DH), F32),
        jax.ShapeDtypeStruct((2 * B, DH), F32),
    ),
    mesh=plsc.VectorSubcoreMesh(core_axis_name="c", subcore_axis_name="s"),
    compiler_params=pltpu.CompilerParams(
        needs_layout_passes=False, use_tc_tiling_on_sc=False),
    scratch_types=(
        pltpu.VMEM_SHARED((NPAD, DH), F32),
        pltpu.VMEM_SHARED((NPAD, DEGW), F32),
        pltpu.VMEM_SHARED((NPAD, DH), F32),
        pltpu.VMEM((NPT, DEGW), F32),
        pltpu.VMEM((NSUB, DH), F32),
        pltpu.VMEM((EK, DH), F32),
        pltpu.VMEM((EK,), I32),
        pltpu.VMEM((EK,), I32),
        pltpu.VMEM((BPT,), I32),
        pltpu.SemaphoreType.DMA,
    ),
)(_sc_body)


def _loss_body(j_ref, p_ref, n_ref, loss_ref, reg_ref):
    jj = j_ref[...]
    pp = p_ref[...]
    nn = n_ref[...]
    dp = jnp.sum(jj * pp, axis=1, keepdims=True)
    dn = jnp.sum(jj * nn, axis=1, keepdims=True)
    ps = dp[:B] + dp[B:]
    ns = dn[:B] + dn[B:]
    d = (ps - ns) * (1.0 / 16.0)
    sig = 1.0 / (1.0 + jnp.exp(-d))
    loss = -jnp.sum(jnp.log(sig + 1e-10)) / B
    reg = (jnp.sum(jj * jj) + jnp.sum(pp * pp) + jnp.sum(nn * nn)) \
        * (1.0 / 16.0) / (2.0 * B)
    loss_ref[...] = jnp.reshape(loss, (1, 1))
    reg_ref[...] = jnp.reshape(reg, (1, 1))


_loss_call = pl.pallas_call(
    _loss_body,
    out_shape=(
        jax.ShapeDtypeStruct((1, 1), F32),
        jax.ShapeDtypeStruct((1, 1), F32),
    ),
)


@jax.jit
def kernel(edge_index, jobs, pos_skills, neg_skills,
           job_table, skill_table, company_table):
    x0 = jnp.concatenate([job_table, skill_table, company_table], axis=0)
    x0p = jnp.pad(x0, ((0, NPAD - N), (0, 0)))
    x0s = x0p.reshape(NPAD, NC, DH).transpose(1, 0, 2).reshape(NC * NPAD, DH)
    row = edge_index[0]
    col = edge_index[1]
    pos_g = pos_skills + N_JOBS
    neg_g = neg_skills + N_JOBS
    _, jg, pg, ng = _sc_kernel(x0s, row, col, jobs, pos_g, neg_g)
    loss, reg = _loss_call(jg, pg, ng)
    return (loss[0, 0], reg[0, 0])


# speedup vs baseline: 1.4435x; 1.4435x over previous
import functools

import jax
import jax.numpy as jnp
from jax import lax
from jax.experimental import pallas as pl
from jax.experimental.pallas import tpu as pltpu
from jax.experimental.pallas import tpu_sc as plsc

N_JOBS = 6000
N_SKILLS = 3000
N_COMP = 1000
N = N_JOBS + N_SKILLS + N_COMP
D = 128
DH = 64
NL = 3
E = 320000
B = 4096

NC = 2
NS = 16
LANES = 16

NPAD = 10240
EPT = E // NS
EK = 200
NECH = EPT // EK
NPT = NPAD // NS
NSUB = 320
BPT = B // NS
BH = 128
DEGW = 16

F32 = jnp.float32
I32 = jnp.int32


def _zero_rows(buf, nrows, ncol16):
    z = jnp.zeros((LANES,), F32)

    def body(i, _):
        for k in range(ncol16):
            buf[i, pl.ds(k * LANES, LANES)] = z
        return 0

    lax.fori_loop(0, nrows, body, 0)


def _add_offset(idx_ref, n, off):
    def body(j, _):
        base = j * LANES
        idx_ref[pl.ds(base, LANES)] = idx_ref[pl.ds(base, LANES)] + off
        return 0

    lax.fori_loop(0, n // LANES, body, 0)


def _sc_body(x0s, row_h, col_h, jobs_h, pos_h, neg_h,
             xp_h, fin_h, jg_h, pg_h, ng_h,
             S, DEG,
             degb, sbuf, fbuf, ridx0, cidx0, ridx1, cidx1, gidx,
             rowsb0, rowsb1, sem0, sem1):
    c = lax.axis_index("c")
    s = lax.axis_index("s")
    nb = s * NPT
    xoff = c * NPAD

    _zero_rows(sbuf, NSUB, DH // LANES)
    _zero_rows(degb, NPT, DEGW // LANES)
    pltpu.sync_copy(degb, DEG.at[pl.ds(nb, NPT)])
    for sub in range(2):
        pltpu.sync_copy(sbuf, S.at[pl.ds(nb + sub * NSUB, NSUB)])
    one16 = jnp.ones((LANES,), F32)

    def fill_ones(i, _):
        degb[i, pl.ds(0, LANES)] = one16
        return 0

    lax.fori_loop(0, NPT, fill_ones, 0)
    plsc.subcore_barrier()

    def deg_chunk(ch, _):
        base = s * EPT + ch * EK
        pltpu.sync_copy(col_h.at[pl.ds(base, EK)], cidx0)
        pltpu.sync_copy(degb.at[pl.ds(0, EK)], DEG.at[cidx0], add=True)
        return 0

    lax.fori_loop(0, NECH, deg_chunk, 0)
    plsc.subcore_barrier()

    pltpu.sync_copy(DEG.at[pl.ds(nb, NPT)], degb)

    def dinv_row(n, _):
        d = degb[n, pl.ds(0, LANES)]
        ib = plsc.bitcast(d, I32)
        ib = jnp.int32(0x5F3759DF) - (ib >> 1)
        y = plsc.bitcast(ib, F32)
        xh = 0.5 * d
        y = y * (1.5 - xh * y * y)
        y = y * (1.5 - xh * y * y)
        y = y * (1.5 - xh * y * y)
        y = jnp.where(d > 0.5, y, 0.0)
        degb[n, pl.ds(0, LANES)] = y
        return 0

    lax.fori_loop(0, NPT, dinv_row, 0)

    for sub in range(2):
        base = nb + sub * NSUB
        pltpu.sync_copy(x0s.at[pl.ds(xoff + base, NSUB)], sbuf)
        pltpu.sync_copy(sbuf, fin_h.at[pl.ds(xoff + base, NSUB)])

        def scale_row(n, _):
            dv = degb[sub * NSUB + n, pl.ds(0, LANES)]
            for k in range(DH // LANES):
                ds = pl.ds(k * LANES, LANES)
                sbuf[n, ds] = dv * sbuf[n, ds]
            return 0

        lax.fori_loop(0, NSUB, scale_row, 0)
        pltpu.sync_copy(sbuf, xp_h.at[pl.ds(xoff + base, NSUB)])
    plsc.subcore_barrier()

    def load_idx(ch, rbuf, cbuf):
        base = s * EPT + ch * EK
        pltpu.sync_copy(row_h.at[pl.ds(base, EK)], rbuf)
        pltpu.sync_copy(col_h.at[pl.ds(base, EK)], cbuf)
        _add_offset(rbuf, EK, xoff)

    for layer in range(NL):
        load_idx(0, ridx0, cidx0)
        pltpu.async_copy(xp_h.at[ridx0], rowsb0, sem0)

        def edge_pair(i, _):
            ch = 2 * i
            load_idx(ch + 1, ridx1, cidx1)
            pltpu.async_copy(xp_h.at[ridx1], rowsb1, sem1)
            pltpu.make_async_copy(xp_h.at[ridx0], rowsb0, sem0).wait()
            pltpu.sync_copy(rowsb0, S.at[cidx0], add=True)
            load_idx((ch + 2) % NECH, ridx0, cidx0)
            pltpu.async_copy(xp_h.at[ridx0], rowsb0, sem0)
            pltpu.make_async_copy(xp_h.at[ridx1], rowsb1, sem1).wait()
            pltpu.sync_copy(rowsb1, S.at[cidx1], add=True)
            return 0

        lax.fori_loop(0, NECH // 2, edge_pair, 0)
        pltpu.make_async_copy(xp_h.at[ridx0], rowsb0, sem0).wait()
        plsc.subcore_barrier()

        for sub in range(2):
            base = nb + sub * NSUB
            pltpu.sync_copy(S.at[pl.ds(base, NSUB)], sbuf)
            pltpu.sync_copy(fin_h.at[pl.ds(xoff + base, NSUB)], fbuf)

            def node_row(n, _):
                dv = degb[sub * NSUB + n, pl.ds(0, LANES)]
                dv2 = dv * dv
                for k in range(DH // LANES):
                    ds = pl.ds(k * LANES, LANES)
                    sl = sbuf[n, ds]
                    fbuf[n, ds] = fbuf[n, ds] + dv * sl
                    sbuf[n, ds] = dv2 * sl
                return 0

            lax.fori_loop(0, NSUB, node_row, 0)
            pltpu.sync_copy(fbuf, fin_h.at[pl.ds(xoff + base, NSUB)])
            if layer < NL - 1:
                pltpu.sync_copy(sbuf, xp_h.at[pl.ds(xoff + base, NSUB)])
                _zero_rows(sbuf, NSUB, DH // LANES)
                pltpu.sync_copy(sbuf, S.at[pl.ds(base, NSUB)])
        plsc.subcore_barrier()

    for idx_h, out_h in ((jobs_h, jg_h), (pos_h, pg_h), (neg_h, ng_h)):
        bb = s * BPT
        pltpu.sync_copy(idx_h.at[pl.ds(bb, BPT)], gidx)
        _add_offset(gidx, BPT, xoff)
        for h in range(BPT // BH):
            pltpu.async_copy(fin_h.at[gidx.at[pl.ds(h * BH, BH)]],
                             rowsb0.at[pl.ds(0, BH)], sem0).wait()
            pltpu.sync_copy(rowsb0.at[pl.ds(0, BH)],
                            out_h.at[pl.ds(c * B + bb + h * BH, BH)])


_sc_kernel = functools.partial(
    pl.kernel,
    out_type=(
        jax.ShapeDtypeStruct((2 * NPAD, DH), F32),
        jax.ShapeDtypeStruct((2 * NPAD, DH), F32),
        jax.ShapeDtypeStruct((2 * B, DH), F32),
        jax.ShapeDtypeStruct((2 * B, DH), F32),
        jax.ShapeDtypeStruct((2 * B, DH), F32),
    ),
    mesh=plsc.VectorSubcoreMesh(core_axis_name="c", subcore_axis_name="s"),
    compiler_params=pltpu.CompilerParams(
        needs_layout_passes=False, use_tc_tiling_on_sc=False),
    scratch_types=(
        pltpu.VMEM_SHARED((NPAD, DH), F32),
        pltpu.VMEM_SHARED((NPAD, DEGW), F32),
        pltpu.VMEM((NPT, DEGW), F32),
        pltpu.VMEM((NSUB, DH), F32),
        pltpu.VMEM((NSUB, DH), F32),
        pltpu.VMEM((EK,), I32),
        pltpu.VMEM((EK,), I32),
        pltpu.VMEM((EK,), I32),
        pltpu.VMEM((EK,), I32),
        pltpu.VMEM((BPT,), I32),
        pltpu.VMEM((EK, DH), F32),
        pltpu.VMEM((EK, DH), F32),
        pltpu.SemaphoreType.DMA,
        pltpu.SemaphoreType.DMA,
    ),
)(_sc_body)


def _loss_body(j_ref, p_ref, n_ref, loss_ref, reg_ref):
    jj = j_ref[...]
    pp = p_ref[...]
    nn = n_ref[...]
    dp = jnp.sum(jj * pp, axis=1, keepdims=True)
    dn = jnp.sum(jj * nn, axis=1, keepdims=True)
    ps = dp[:B] + dp[B:]
    ns = dn[:B] + dn[B:]
    d = (ps - ns) * (1.0 / 16.0)
    sig = 1.0 / (1.0 + jnp.exp(-d))
    loss = -jnp.sum(jnp.log(sig + 1e-10)) / B
    reg = (jnp.sum(jj * jj) + jnp.sum(pp * pp) + jnp.sum(nn * nn)) \
        * (1.0 / 16.0) / (2.0 * B)
    loss_ref[...] = jnp.reshape(loss, (1, 1))
    reg_ref[...] = jnp.reshape(reg, (1, 1))


_loss_call = pl.pallas_call(
    _loss_body,
    out_shape=(
        jax.ShapeDtypeStruct((1, 1), F32),
        jax.ShapeDtypeStruct((1, 1), F32),
    ),
)


@jax.jit
def kernel(edge_index, jobs, pos_skills, neg_skills,
           job_table, skill_table, company_table):
    x0 = jnp.concatenate([job_table, skill_table, company_table], axis=0)
    x0p = jnp.pad(x0, ((0, NPAD - N), (0, 0)))
    x0s = x0p.reshape(NPAD, NC, DH).transpose(1, 0, 2).reshape(NC * NPAD, DH)
    row = edge_index[0]
    col = edge_index[1]
    pos_g = pos_skills + N_JOBS
    neg_g = neg_skills + N_JOBS
    _, _, jg, pg, ng = _sc_kernel(x0s, row, col, jobs, pos_g, neg_g)
    loss, reg = _loss_call(jg, pg, ng)
    return (loss[0, 0], reg[0, 0])


# speedup vs baseline: 1.8498x; 1.2815x over previous
import functools

import jax
import jax.numpy as jnp
from jax import lax
from jax.experimental import pallas as pl
from jax.experimental.pallas import tpu as pltpu
from jax.experimental.pallas import tpu_sc as plsc

N_JOBS = 6000
N_SKILLS = 3000
N_COMP = 1000
N = N_JOBS + N_SKILLS + N_COMP
D = 128
DH = 64
NL = 3
E = 320000
B = 4096

NC = 2
NS = 16
LANES = 16

NPAD = 10240
EPT = E // NS
EK = 200
NECH = EPT // EK
NPT = NPAD // NS
NSUB = 320
BPT = B // NS
BH = 128
DEGW = 16

F32 = jnp.float32
I32 = jnp.int32


def _zero_rows(buf, nrows, ncol16):
    z = jnp.zeros((LANES,), F32)

    def body(i, _):
        for k in range(ncol16):
            buf[i, pl.ds(k * LANES, LANES)] = z
        return 0

    lax.fori_loop(0, nrows, body, 0)


def _add_offset(idx_ref, n, off):
    def body(j, _):
        base = j * LANES
        idx_ref[pl.ds(base, LANES)] = idx_ref[pl.ds(base, LANES)] + off
        return 0

    lax.fori_loop(0, n // LANES, body, 0)


def _sc_body(x0s, eix_h, jobs_h, pos_h, neg_h,
             xp_h, fin_h, jg_h, pg_h, ng_h,
             S, DEG,
             degb, sbuf, fbuf, ebuf0, ebuf1, gidx,
             rowsb0, rowsb1, sem0, sem1, semi0, semi1):
    c = lax.axis_index("c")
    s = lax.axis_index("s")
    nb = s * NPT
    xoff = c * NPAD

    _zero_rows(sbuf, NSUB, DH // LANES)
    _zero_rows(degb, NPT, DEGW // LANES)
    pltpu.sync_copy(degb, DEG.at[pl.ds(nb, NPT)])
    for sub in range(2):
        pltpu.sync_copy(sbuf, S.at[pl.ds(nb + sub * NSUB, NSUB)])
    one16 = jnp.ones((LANES,), F32)

    def fill_ones(i, _):
        degb[i, pl.ds(0, LANES)] = one16
        return 0

    lax.fori_loop(0, NPT, fill_ones, 0)
    plsc.subcore_barrier()

    def col_load(ch, buf, sem):
        base = s * EPT + ch * EK
        return pltpu.async_copy(eix_h.at[1, pl.ds(base, EK)], buf.at[1], sem)

    col_load(0, ebuf0, semi0).wait()

    def deg_pair(i, _):
        ch = 2 * i
        col_load(ch + 1, ebuf1, semi1)
        pltpu.sync_copy(degb.at[pl.ds(0, EK)], DEG.at[ebuf0.at[1]], add=True)
        col_load((ch + 2) % NECH, ebuf0, semi0)
        pltpu.make_async_copy(eix_h.at[1, pl.ds(0, EK)], ebuf1.at[1],
                              semi1).wait()
        pltpu.sync_copy(degb.at[pl.ds(0, EK)], DEG.at[ebuf1.at[1]], add=True)
        pltpu.make_async_copy(eix_h.at[1, pl.ds(0, EK)], ebuf0.at[1],
                              semi0).wait()
        return 0

    lax.fori_loop(0, NECH // 2, deg_pair, 0)
    plsc.subcore_barrier()

    pltpu.sync_copy(DEG.at[pl.ds(nb, NPT)], degb)

    def dinv_row(n, _):
        d = degb[n, pl.ds(0, LANES)]
        ib = plsc.bitcast(d, I32)
        ib = jnp.int32(0x5F3759DF) - (ib >> 1)
        y = plsc.bitcast(ib, F32)
        xh = 0.5 * d
        y = y * (1.5 - xh * y * y)
        y = y * (1.5 - xh * y * y)
        y = y * (1.5 - xh * y * y)
        y = jnp.where(d > 0.5, y, 0.0)
        degb[n, pl.ds(0, LANES)] = y
        return 0

    lax.fori_loop(0, NPT, dinv_row, 0)

    for sub in range(2):
        base = nb + sub * NSUB
        pltpu.sync_copy(x0s.at[pl.ds(xoff + base, NSUB)], sbuf)
        pltpu.sync_copy(sbuf, fin_h.at[pl.ds(xoff + base, NSUB)])

        def scale_row(n, _):
            dv = degb[sub * NSUB + n, pl.ds(0, LANES)]
            for k in range(DH // LANES):
                ds = pl.ds(k * LANES, LANES)
                sbuf[n, ds] = dv * sbuf[n, ds]
            return 0

        lax.fori_loop(0, NSUB, scale_row, 0)
        pltpu.sync_copy(sbuf, xp_h.at[pl.ds(xoff + base, NSUB)])
    plsc.subcore_barrier()

    def idx_load(ch, buf, sem):
        base = s * EPT + ch * EK
        return pltpu.async_copy(eix_h.at[:, pl.ds(base, EK)], buf, sem)

    def start_gather(buf, rows, sem):
        _add_offset(buf.at[0], EK, xoff)
        return pltpu.async_copy(xp_h.at[buf.at[0]], rows, sem)

    def wait_gather(buf, rows, sem):
        pltpu.make_async_copy(xp_h.at[buf.at[0]], rows, sem).wait()

    def wait_idx(buf, sem):
        pltpu.make_async_copy(eix_h.at[:, pl.ds(0, EK)], buf, sem).wait()

    for layer in range(NL):
        idx_load(0, ebuf0, semi0).wait()
        start_gather(ebuf0, rowsb0, sem0)
        idx_load(1, ebuf1, semi1)

        def edge_pair(i, _):
            ch = 2 * i
            wait_idx(ebuf1, semi1)
            start_gather(ebuf1, rowsb1, sem1)
            wait_gather(ebuf0, rowsb0, sem0)
            pltpu.sync_copy(rowsb0, S.at[ebuf0.at[1]], add=True)
            idx_load((ch + 2) % NECH, ebuf0, semi0)
            wait_idx(ebuf0, semi0)
            start_gather(ebuf0, rowsb0, sem0)
            wait_gather(ebuf1, rowsb1, sem1)
            pltpu.sync_copy(rowsb1, S.at[ebuf1.at[1]], add=True)
            idx_load((ch + 3) % NECH, ebuf1, semi1)
            return 0

        lax.fori_loop(0, NECH // 2, edge_pair, 0)
        wait_gather(ebuf0, rowsb0, sem0)
        wait_idx(ebuf1, semi1)
        plsc.subcore_barrier()

        for sub in range(2):
            base = nb + sub * NSUB
            pltpu.sync_copy(S.at[pl.ds(base, NSUB)], sbuf)
            pltpu.sync_copy(fin_h.at[pl.ds(xoff + base, NSUB)], fbuf)

            def node_row(n, _):
                dv = degb[sub * NSUB + n, pl.ds(0, LANES)]
                dv2 = dv * dv
                for k in range(DH // LANES):
                    ds = pl.ds(k * LANES, LANES)
                    sl = sbuf[n, ds]
                    fbuf[n, ds] = fbuf[n, ds] + dv * sl
                    sbuf[n, ds] = dv2 * sl
                return 0

            lax.fori_loop(0, NSUB, node_row, 0)
            pltpu.sync_copy(fbuf, fin_h.at[pl.ds(xoff + base, NSUB)])
            if layer < NL - 1:
                pltpu.sync_copy(sbuf, xp_h.at[pl.ds(xoff + base, NSUB)])
                _zero_rows(sbuf, NSUB, DH // LANES)
                pltpu.sync_copy(sbuf, S.at[pl.ds(base, NSUB)])
        plsc.subcore_barrier()

    for idx_h, out_h in ((jobs_h, jg_h), (pos_h, pg_h), (neg_h, ng_h)):
        bb = s * BPT
        pltpu.sync_copy(idx_h.at[pl.ds(bb, BPT)], gidx)
        _add_offset(gidx, BPT, xoff)
        for h in range(BPT // BH):
            pltpu.async_copy(fin_h.at[gidx.at[pl.ds(h * BH, BH)]],
                             rowsb0.at[pl.ds(0, BH)], sem0).wait()
            pltpu.sync_copy(rowsb0.at[pl.ds(0, BH)],
                            out_h.at[pl.ds(c * B + bb + h * BH, BH)])


_sc_kernel = functools.partial(
    pl.kernel,
    out_type=(
        jax.ShapeDtypeStruct((2 * NPAD, DH), F32),
        jax.ShapeDtypeStruct((2 * NPAD, DH), F32),
        jax.ShapeDtypeStruct((2 * B, DH), F32),
        jax.ShapeDtypeStruct((2 * B, DH), F32),
        jax.ShapeDtypeStruct((2 * B, DH), F32),
    ),
    mesh=plsc.VectorSubcoreMesh(core_axis_name="c", subcore_axis_name="s"),
    compiler_params=pltpu.CompilerParams(
        needs_layout_passes=False, use_tc_tiling_on_sc=False),
    scratch_types=(
        pltpu.VMEM_SHARED((NPAD, DH), F32),
        pltpu.VMEM_SHARED((NPAD, DEGW), F32),
        pltpu.VMEM((NPT, DEGW), F32),
        pltpu.VMEM((NSUB, DH), F32),
        pltpu.VMEM((NSUB, DH), F32),
        pltpu.VMEM((2, EK), I32),
        pltpu.VMEM((2, EK), I32),
        pltpu.VMEM((BPT,), I32),
        pltpu.VMEM((EK, DH), F32),
        pltpu.VMEM((EK, DH), F32),
        pltpu.SemaphoreType.DMA,
        pltpu.SemaphoreType.DMA,
        pltpu.SemaphoreType.DMA,
        pltpu.SemaphoreType.DMA,
    ),
)(_sc_body)


def _loss_body(j_ref, p_ref, n_ref, loss_ref, reg_ref):
    jj = j_ref[...]
    pp = p_ref[...]
    nn = n_ref[...]
    dp = jnp.sum(jj * pp, axis=1, keepdims=True)
    dn = jnp.sum(jj * nn, axis=1, keepdims=True)
    ps = dp[:B] + dp[B:]
    ns = dn[:B] + dn[B:]
    d = (ps - ns) * (1.0 / 16.0)
    sig = 1.0 / (1.0 + jnp.exp(-d))
    loss = -jnp.sum(jnp.log(sig + 1e-10)) / B
    reg = (jnp.sum(jj * jj) + jnp.sum(pp * pp) + jnp.sum(nn * nn)) \
        * (1.0 / 16.0) / (2.0 * B)
    loss_ref[...] = jnp.reshape(loss, (1, 1))
    reg_ref[...] = jnp.reshape(reg, (1, 1))


_loss_call = pl.pallas_call(
    _loss_body,
    out_shape=(
        jax.ShapeDtypeStruct((1, 1), F32),
        jax.ShapeDtypeStruct((1, 1), F32),
    ),
)


@jax.jit
def kernel(edge_index, jobs, pos_skills, neg_skills,
           job_table, skill_table, company_table):
    x0 = jnp.concatenate([job_table, skill_table, company_table], axis=0)
    x0p = jnp.pad(x0, ((0, NPAD - N), (0, 0)))
    x0s = x0p.reshape(NPAD, NC, DH).transpose(1, 0, 2).reshape(NC * NPAD, DH)
    pos_g = pos_skills + N_JOBS
    neg_g = neg_skills + N_JOBS
    _, _, jg, pg, ng = _sc_kernel(x0s, edge_index, jobs, pos_g, neg_g)
    loss, reg = _loss_call(jg, pg, ng)
    return (loss[0, 0], reg[0, 0])


# speedup vs baseline: 2.1018x; 1.1362x over previous
import functools

import jax
import jax.numpy as jnp
from jax import lax
from jax.experimental import pallas as pl
from jax.experimental.pallas import tpu as pltpu
from jax.experimental.pallas import tpu_sc as plsc

N_JOBS = 6000
N_SKILLS = 3000
N_COMP = 1000
N = N_JOBS + N_SKILLS + N_COMP
D = 128
DH = 64
NL = 3
E = 320000
B = 4096

NC = 2
NS = 16
LANES = 16

NPAD = 10240
EPT = E // NS
EK = 400
NECH = EPT // EK
NPT = NPAD // NS
NSUB = 160
NSC = NPT // NSUB
BPT = B // NS
BH = 128
DEGW = 16

F32 = jnp.float32
I32 = jnp.int32


def _zero_rows(buf, nrows, ncol16):
    z = jnp.zeros((LANES,), F32)

    def body(i, _):
        for k in range(ncol16):
            buf[i, pl.ds(k * LANES, LANES)] = z
        return 0

    lax.fori_loop(0, nrows, body, 0)


def _add_offset(idx_ref, n, off):
    def body(j, _):
        base = j * LANES
        idx_ref[pl.ds(base, LANES)] = idx_ref[pl.ds(base, LANES)] + off
        return 0

    lax.fori_loop(0, n // LANES, body, 0)


def _sc_body(x0s, eix_h, jobs_h, pos_h, neg_h,
             xp_h, fin_h, jg_h, pg_h, ng_h,
             S, DEG,
             degb, ebuf0, ebuf1, gidx,
             rowsb0, rowsb1, sem0, sem1, semi0, semi1):
    sbuf = rowsb0.at[pl.ds(0, NSUB)]
    fbuf = rowsb1.at[pl.ds(0, NSUB)]
    c = lax.axis_index("c")
    s = lax.axis_index("s")
    nb = s * NPT
    xoff = c * NPAD

    _zero_rows(sbuf, NSUB, DH // LANES)
    _zero_rows(degb, NPT, DEGW // LANES)
    pltpu.sync_copy(degb, DEG.at[pl.ds(nb, NPT)])
    for sub in range(NSC):
        pltpu.sync_copy(sbuf, S.at[pl.ds(nb + sub * NSUB, NSUB)])
    one16 = jnp.ones((LANES,), F32)

    def fill_ones(i, _):
        degb[i, pl.ds(0, LANES)] = one16
        return 0

    lax.fori_loop(0, NPT, fill_ones, 0)
    plsc.subcore_barrier()

    def col_load(ch, buf, sem):
        base = s * EPT + ch * EK
        return pltpu.async_copy(eix_h.at[1, pl.ds(base, EK)], buf.at[1], sem)

    col_load(0, ebuf0, semi0).wait()

    def deg_pair(i, _):
        ch = 2 * i
        col_load(ch + 1, ebuf1, semi1)
        pltpu.sync_copy(degb.at[pl.ds(0, EK)], DEG.at[ebuf0.at[1]], add=True)
        col_load((ch + 2) % NECH, ebuf0, semi0)
        pltpu.make_async_copy(eix_h.at[1, pl.ds(0, EK)], ebuf1.at[1],
                              semi1).wait()
        pltpu.sync_copy(degb.at[pl.ds(0, EK)], DEG.at[ebuf1.at[1]], add=True)
        pltpu.make_async_copy(eix_h.at[1, pl.ds(0, EK)], ebuf0.at[1],
                              semi0).wait()
        return 0

    lax.fori_loop(0, NECH // 2, deg_pair, 0)
    plsc.subcore_barrier()

    pltpu.sync_copy(DEG.at[pl.ds(nb, NPT)], degb)

    def dinv_row(n, _):
        d = degb[n, pl.ds(0, LANES)]
        ib = plsc.bitcast(d, I32)
        ib = jnp.int32(0x5F3759DF) - (ib >> 1)
        y = plsc.bitcast(ib, F32)
        xh = 0.5 * d
        y = y * (1.5 - xh * y * y)
        y = y * (1.5 - xh * y * y)
        y = y * (1.5 - xh * y * y)
        y = jnp.where(d > 0.5, y, 0.0)
        degb[n, pl.ds(0, LANES)] = y
        return 0

    lax.fori_loop(0, NPT, dinv_row, 0)

    for sub in range(NSC):
        base = nb + sub * NSUB
        pltpu.sync_copy(x0s.at[pl.ds(xoff + base, NSUB)], sbuf)
        pltpu.sync_copy(sbuf, fin_h.at[pl.ds(xoff + base, NSUB)])

        def scale_row(n, _):
            dv = degb[sub * NSUB + n, pl.ds(0, LANES)]
            for k in range(DH // LANES):
                ds = pl.ds(k * LANES, LANES)
                sbuf[n, ds] = dv * sbuf[n, ds]
            return 0

        lax.fori_loop(0, NSUB, scale_row, 0)
        pltpu.sync_copy(sbuf, xp_h.at[pl.ds(xoff + base, NSUB)])
    plsc.subcore_barrier()

    def idx_load(ch, buf, sem):
        base = s * EPT + ch * EK
        return pltpu.async_copy(eix_h.at[:, pl.ds(base, EK)], buf, sem)

    def start_gather(buf, rows, sem):
        _add_offset(buf.at[0], EK, xoff)
        return pltpu.async_copy(xp_h.at[buf.at[0]], rows, sem)

    def wait_gather(buf, rows, sem):
        pltpu.make_async_copy(xp_h.at[buf.at[0]], rows, sem).wait()

    def wait_idx(buf, sem):
        pltpu.make_async_copy(eix_h.at[:, pl.ds(0, EK)], buf, sem).wait()

    for layer in range(NL):
        idx_load(0, ebuf0, semi0).wait()
        start_gather(ebuf0, rowsb0, sem0)
        idx_load(1, ebuf1, semi1)

        def edge_pair(i, _):
            ch = 2 * i
            wait_idx(ebuf1, semi1)
            start_gather(ebuf1, rowsb1, sem1)
            wait_gather(ebuf0, rowsb0, sem0)
            pltpu.sync_copy(rowsb0, S.at[ebuf0.at[1]], add=True)
            idx_load((ch + 2) % NECH, ebuf0, semi0)
            wait_idx(ebuf0, semi0)
            start_gather(ebuf0, rowsb0, sem0)
            wait_gather(ebuf1, rowsb1, sem1)
            pltpu.sync_copy(rowsb1, S.at[ebuf1.at[1]], add=True)
            idx_load((ch + 3) % NECH, ebuf1, semi1)
            return 0

        lax.fori_loop(0, NECH // 2, edge_pair, 0)
        wait_gather(ebuf0, rowsb0, sem0)
        wait_idx(ebuf1, semi1)
        plsc.subcore_barrier()

        for sub in range(NSC):
            base = nb + sub * NSUB
            pltpu.sync_copy(S.at[pl.ds(base, NSUB)], sbuf)
            pltpu.sync_copy(fin_h.at[pl.ds(xoff + base, NSUB)], fbuf)

            def node_row(n, _):
                dv = degb[sub * NSUB + n, pl.ds(0, LANES)]
                dv2 = dv * dv
                for k in range(DH // LANES):
                    ds = pl.ds(k * LANES, LANES)
                    sl = sbuf[n, ds]
                    fbuf[n, ds] = fbuf[n, ds] + dv * sl
                    sbuf[n, ds] = dv2 * sl
                return 0

            lax.fori_loop(0, NSUB, node_row, 0)
            pltpu.sync_copy(fbuf, fin_h.at[pl.ds(xoff + base, NSUB)])
            if layer < NL - 1:
                pltpu.sync_copy(sbuf, xp_h.at[pl.ds(xoff + base, NSUB)])
                _zero_rows(sbuf, NSUB, DH // LANES)
                pltpu.sync_copy(sbuf, S.at[pl.ds(base, NSUB)])
        plsc.subcore_barrier()

    for idx_h, out_h in ((jobs_h, jg_h), (pos_h, pg_h), (neg_h, ng_h)):
        bb = s * BPT
        pltpu.sync_copy(idx_h.at[pl.ds(bb, BPT)], gidx)
        _add_offset(gidx, BPT, xoff)
        for h in range(BPT // BH):
            pltpu.async_copy(fin_h.at[gidx.at[pl.ds(h * BH, BH)]],
                             rowsb0.at[pl.ds(0, BH)], sem0).wait()
            pltpu.sync_copy(rowsb0.at[pl.ds(0, BH)],
                            out_h.at[pl.ds(c * B + bb + h * BH, BH)])


_sc_kernel = functools.partial(
    pl.kernel,
    out_type=(
        jax.ShapeDtypeStruct((2 * NPAD, DH), F32),
        jax.ShapeDtypeStruct((2 * NPAD, DH), F32),
        jax.ShapeDtypeStruct((2 * B, DH), F32),
        jax.ShapeDtypeStruct((2 * B, DH), F32),
        jax.ShapeDtypeStruct((2 * B, DH), F32),
    ),
    mesh=plsc.VectorSubcoreMesh(core_axis_name="c", subcore_axis_name="s"),
    compiler_params=pltpu.CompilerParams(
        needs_layout_passes=False, use_tc_tiling_on_sc=False),
    scratch_types=(
        pltpu.VMEM_SHARED((NPAD, DH), F32),
        pltpu.VMEM_SHARED((NPAD, DEGW), F32),
        pltpu.VMEM((NPT, DEGW), F32),
        pltpu.VMEM((2, EK), I32),
        pltpu.VMEM((2, EK), I32),
        pltpu.VMEM((BPT,), I32),
        pltpu.VMEM((EK, DH), F32),
        pltpu.VMEM((EK, DH), F32),
        pltpu.SemaphoreType.DMA,
        pltpu.SemaphoreType.DMA,
        pltpu.SemaphoreType.DMA,
        pltpu.SemaphoreType.DMA,
    ),
)(_sc_body)


def _loss_body(j_ref, p_ref, n_ref, loss_ref, reg_ref):
    jj = j_ref[...]
    pp = p_ref[...]
    nn = n_ref[...]
    dp = jnp.sum(jj * pp, axis=1, keepdims=True)
    dn = jnp.sum(jj * nn, axis=1, keepdims=True)
    ps = dp[:B] + dp[B:]
    ns = dn[:B] + dn[B:]
    d = (ps - ns) * (1.0 / 16.0)
    sig = 1.0 / (1.0 + jnp.exp(-d))
    loss = -jnp.sum(jnp.log(sig + 1e-10)) / B
    reg = (jnp.sum(jj * jj) + jnp.sum(pp * pp) + jnp.sum(nn * nn)) \
        * (1.0 / 16.0) / (2.0 * B)
    loss_ref[...] = jnp.reshape(loss, (1, 1))
    reg_ref[...] = jnp.reshape(reg, (1, 1))


_loss_call = pl.pallas_call(
    _loss_body,
    out_shape=(
        jax.ShapeDtypeStruct((1, 1), F32),
        jax.ShapeDtypeStruct((1, 1), F32),
    ),
)


@jax.jit
def kernel(edge_index, jobs, pos_skills, neg_skills,
           job_table, skill_table, company_table):
    x0 = jnp.concatenate([job_table, skill_table, company_table], axis=0)
    x0p = jnp.pad(x0, ((0, NPAD - N), (0, 0)))
    x0s = x0p.reshape(NPAD, NC, DH).transpose(1, 0, 2).reshape(NC * NPAD, DH)
    pos_g = pos_skills + N_JOBS
    neg_g = neg_skills + N_JOBS
    _, _, jg, pg, ng = _sc_kernel(x0s, edge_index, jobs, pos_g, neg_g)
    loss, reg = _loss_call(jg, pg, ng)
    return (loss[0, 0], reg[0, 0])
